# baseline (speedup 1.0000x reference)
import functools

import jax
import jax.numpy as jnp
from jax.experimental import pallas as pl
from jax.experimental.pallas import tpu as pltpu

N_NODES = 4096
N_EDGES_H = 2048
NNZ = 65536
N_GRAPHS = 64
D = 128
M_DIM = 16
K_NN = 16
VALID_RADIUS = 5.0
N_LAYERS = 3
NUM_ATOM_FEATS = 9


def _ln(x, g, b, eps=1e-5):
    mu = x.mean(-1, keepdims=True)
    var = ((x - mu) ** 2).mean(-1, keepdims=True)
    return (x - mu) / jnp.sqrt(var + eps) * g + b


def _mlp2(x, p, nm):
    h = x @ p[nm + '_Wa'] + p[nm + '_ba']
    h = jax.nn.relu(_ln(h, p[nm + '_g'], p[nm + '_b']))
    return h @ p[nm + '_Wb'] + p[nm + '_bb']


def _egnn(feats, coors, p):
    sq = (coors ** 2).sum(-1)
    dist2 = sq[:, None] + sq[None, :] - 2.0 * (coors @ coors.T)
    neg, idx = jax.lax.top_k(-dist2, K_NN)
    d_k = jnp.maximum(-neg, 0.0)
    feats_j = feats[idx]
    feats_i = jnp.broadcast_to(feats[:, None, :], feats_j.shape)
    edge_in = jnp.concatenate([feats_i, feats_j, d_k[..., None]], axis=-1)
    m = jax.nn.silu(edge_in @ p['We1'] + p['be1'])
    m = jax.nn.silu(m @ p['We2'] + p['be2'])
    mask = d_k <= VALID_RADIUS ** 2
    m_i = jnp.where(mask[..., None], m, 0.0).sum(axis=1)
    node_in = jnp.concatenate([_ln(feats, p['ng'], p['nb']), m_i], axis=-1)
    h = jax.nn.silu(node_in @ p['Wn1'] + p['bn1']) @ p['Wn2'] + p['bn2']
    return feats + h


def _conv(X, V, E, X0, p):
    Xve = _mlp2(X, p, 'c1')[V]
    Xe = jax.nn.relu(jax.ops.segment_sum(Xve, E, num_segments=N_EDGES_H))
    Xev = _mlp2(Xe, p, 'c2')[E]
    Xv = jax.ops.segment_sum(Xev, V, num_segments=N_NODES)
    return _mlp2(Xv + X0, p, 'c3')


def _out_mlp_kernel(pooled_ref, wa_ref, ba_ref, g_ref, b_ref, wb_ref, bb_ref,
                    out_ref):
    h = jnp.dot(pooled_ref[...], wa_ref[...],
                preferred_element_type=jnp.float32) + ba_ref[...]
    mu = h.mean(-1, keepdims=True)
    var = ((h - mu) ** 2).mean(-1, keepdims=True)
    h = (h - mu) / jnp.sqrt(var + 1e-5) * g_ref[...] + b_ref[...]
    h = jax.nn.relu(h)
    out_ref[...] = jnp.dot(h, wb_ref[...],
                           preferred_element_type=jnp.float32) + bb_ref[...]


def _out_mlp(pooled, p):
    out = pl.pallas_call(
        _out_mlp_kernel,
        out_shape=jax.ShapeDtypeStruct((N_GRAPHS, 1), jnp.float32),
    )(pooled, p['out_Wa'], p['out_ba'].reshape(1, D), p['out_g'].reshape(1, D),
      p['out_b'].reshape(1, D), p['out_Wb'], p['out_bb'].reshape(1, 1))
    return out


def kernel(x, pos, edge_index0, edge_index1, batch, params):
    emb = params['atom_emb']
    feats = jnp.zeros((x.shape[0], D), jnp.float32)
    for f in range(NUM_ATOM_FEATS):
        feats = feats + emb[f][x[:, f]]
    feats = _egnn(feats, pos, params)
    x0 = feats
    h = feats
    for _ in range(N_LAYERS):
        h = jax.nn.relu(_conv(h, edge_index0, edge_index1, x0, params))
    pooled = jax.ops.segment_sum(h, batch, num_segments=N_GRAPHS)
    out = _out_mlp(pooled, params)
    return out.reshape(-1)


# speedup vs baseline: 1.5543x; 1.5543x over previous
import functools

import jax
import jax.numpy as jnp
from jax.experimental import pallas as pl
from jax.experimental.pallas import tpu as pltpu

N_NODES = 4096
N_EDGES_H = 2048
NNZ = 65536
N_GRAPHS = 64
D = 128
M_DIM = 16
K_NN = 16
VALID_RADIUS = 5.0
N_LAYERS = 3
NUM_ATOM_FEATS = 9
ATOM_VOCAB = 119

EI = 2 * D + 1
EH = EI * 2
NB = 256
EB = NB * K_NN


def _ln_rows(h, g, b, eps=1e-5):
    mu = h.mean(-1, keepdims=True)
    var = ((h - mu) ** 2).mean(-1, keepdims=True)
    return (h - mu) / jnp.sqrt(var + eps) * g + b


def _emb_kernel(x_ref, emb_ref, out_ref):
    xb = x_ref[...]
    lanes = jax.lax.broadcasted_iota(jnp.int32, (xb.shape[0], 9 * D), 1)
    f_of_lane = lanes >> 7
    v_of_lane = lanes & 127
    x_sel = jnp.zeros_like(lanes)
    for f in range(NUM_ATOM_FEATS):
        x_sel = jnp.where(f_of_lane == f, xb[:, f][:, None], x_sel)
    oh = (x_sel == v_of_lane).astype(jnp.float32)
    out_ref[...] = jax.lax.dot(oh, emb_ref[...],
                               preferred_element_type=jnp.float32)


def _embed(x, emb_pad):
    nbe = 512
    return pl.pallas_call(
        _emb_kernel,
        grid=(N_NODES // nbe,),
        in_specs=[
            pl.BlockSpec((nbe, NUM_ATOM_FEATS), lambda i: (i, 0)),
            pl.BlockSpec((NUM_ATOM_FEATS * D, D), lambda i: (0, 0)),
        ],
        out_specs=pl.BlockSpec((nbe, D), lambda i: (i, 0)),
        out_shape=jax.ShapeDtypeStruct((N_NODES, D), jnp.float32),
    )(x, emb_pad)


def _egnn_kernel(feats_ref, fj_ref, dk_ref,
                 we1a_ref, we1b_ref, we1d_ref, be1_ref, we2_ref, be2_ref,
                 ng_ref, nb_ref, wn1a_ref, wn1b_ref, bn1_ref, wn2_ref,
                 bn2_ref, out_ref):
    feats = feats_ref[...]
    dcol = dk_ref[...]
    f1 = jax.lax.dot(feats, we1a_ref[...],
                     preferred_element_type=jnp.float32)
    fj = fj_ref[...]
    f2 = jax.lax.dot(fj, we1b_ref[...],
                     preferred_element_type=jnp.float32)
    m1 = f2 + dcol * we1d_ref[...] + be1_ref[...]
    m1 = (m1.reshape(NB, K_NN, EH) + f1[:, None, :]).reshape(EB, EH)
    m1 = m1 * jax.nn.sigmoid(m1)
    m2 = jax.lax.dot(m1, we2_ref[...],
                     preferred_element_type=jnp.float32) + be2_ref[...]
    m = m2 * jax.nn.sigmoid(m2)
    m = jnp.where(dcol <= VALID_RADIUS ** 2, m, 0.0)
    rows = jax.lax.broadcasted_iota(jnp.int32, (NB, EB), 0)
    cols = jax.lax.broadcasted_iota(jnp.int32, (NB, EB), 1)
    smat = ((cols >> 4) == rows).astype(jnp.float32)
    m_i = jax.lax.dot(smat, m, preferred_element_type=jnp.float32)
    ln_f = _ln_rows(feats, ng_ref[...], nb_ref[...])
    h = (jax.lax.dot(ln_f, wn1a_ref[...], preferred_element_type=jnp.float32)
         + jax.lax.dot(m_i, wn1b_ref[...], preferred_element_type=jnp.float32)
         + bn1_ref[...])
    h = h * jax.nn.sigmoid(h)
    h = jax.lax.dot(h, wn2_ref[...],
                    preferred_element_type=jnp.float32) + bn2_ref[...]
    out_ref[...] = feats + h


def _egnn_dense(feats, feats_j, d_k, p):
    grid = N_NODES // NB
    rep = lambda i: (0, 0)
    return pl.pallas_call(
        _egnn_kernel,
        grid=(grid,),
        in_specs=[
            pl.BlockSpec((NB, D), lambda i: (i, 0)),
            pl.BlockSpec((EB, D), lambda i: (i, 0)),
            pl.BlockSpec((EB, 1), lambda i: (i, 0)),
            pl.BlockSpec((D, EH), rep),
            pl.BlockSpec((D, EH), rep),
            pl.BlockSpec((1, EH), rep),
            pl.BlockSpec((1, EH), rep),
            pl.BlockSpec((EH, M_DIM), rep),
            pl.BlockSpec((1, M_DIM), rep),
            pl.BlockSpec((1, D), rep),
            pl.BlockSpec((1, D), rep),
            pl.BlockSpec((D, 2 * D), rep),
            pl.BlockSpec((M_DIM, 2 * D), rep),
            pl.BlockSpec((1, 2 * D), rep),
            pl.BlockSpec((2 * D, D), rep),
            pl.BlockSpec((1, D), rep),
        ],
        out_specs=pl.BlockSpec((NB, D), lambda i: (i, 0)),
        out_shape=jax.ShapeDtypeStruct((N_NODES, D), jnp.float32),
    )(feats, feats_j, d_k,
      p['We1'][:D], p['We1'][D:2 * D], p['We1'][2 * D:2 * D + 1],
      p['be1'].reshape(1, EH), p['We2'], p['be2'].reshape(1, M_DIM),
      p['ng'].reshape(1, D), p['nb'].reshape(1, D),
      p['Wn1'][:D], p['Wn1'][D:], p['bn1'].reshape(1, 2 * D),
      p['Wn2'], p['bn2'].reshape(1, D))


def _mlp2_kernel(x_ref, wa_ref, ba_ref, g_ref, b_ref, wb_ref, bb_ref,
                 out_ref, *, relu_out):
    h = jax.lax.dot(x_ref[...], wa_ref[...],
                    preferred_element_type=jnp.float32) + ba_ref[...]
    h = jax.nn.relu(_ln_rows(h, g_ref[...], b_ref[...]))
    o = jax.lax.dot(h, wb_ref[...],
                    preferred_element_type=jnp.float32) + bb_ref[...]
    if relu_out:
        o = jax.nn.relu(o)
    out_ref[...] = o


def _mlp2_pallas(x, p, nm, relu_out=False, block=512):
    n = x.shape[0]
    out_d = p[nm + '_Wb'].shape[1]
    rep = lambda i: (0, 0)
    return pl.pallas_call(
        functools.partial(_mlp2_kernel, relu_out=relu_out),
        grid=(n // block,),
        in_specs=[
            pl.BlockSpec((block, D), lambda i: (i, 0)),
            pl.BlockSpec((D, D), rep),
            pl.BlockSpec((1, D), rep),
            pl.BlockSpec((1, D), rep),
            pl.BlockSpec((1, D), rep),
            pl.BlockSpec((D, out_d), rep),
            pl.BlockSpec((1, out_d), rep),
        ],
        out_specs=pl.BlockSpec((block, out_d), lambda i: (i, 0)),
        out_shape=jax.ShapeDtypeStruct((n, out_d), jnp.float32),
    )(x, p[nm + '_Wa'], p[nm + '_ba'].reshape(1, D), p[nm + '_g'].reshape(1, D),
      p[nm + '_b'].reshape(1, D), p[nm + '_Wb'],
      p[nm + '_bb'].reshape(1, out_d))


def _spmm_kernel(b_ref, y_ref, out_ref, *, relu_out, k_steps):
    @pl.when(pl.program_id(1) == 0)
    def _init():
        out_ref[...] = jnp.zeros_like(out_ref)
    out_ref[...] += jax.lax.dot(b_ref[...], y_ref[...],
                                preferred_element_type=jnp.float32)
    if relu_out:
        @pl.when(pl.program_id(1) == k_steps - 1)
        def _fin():
            out_ref[...] = jax.nn.relu(out_ref[...])


def _spmm(b, y, relu_out, mb=512, kb=2048):
    m, k = b.shape
    d = y.shape[1]
    k_steps = k // kb
    return pl.pallas_call(
        functools.partial(_spmm_kernel, relu_out=relu_out, k_steps=k_steps),
        grid=(m // mb, k_steps),
        in_specs=[
            pl.BlockSpec((mb, kb), lambda i, j: (i, j)),
            pl.BlockSpec((kb, d), lambda i, j: (j, 0)),
        ],
        out_specs=pl.BlockSpec((mb, d), lambda i, j: (i, 0)),
        out_shape=jax.ShapeDtypeStruct((m, d), jnp.float32),
    )(b, y)


def _pool_out_kernel(h_ref, batch_ref, wa_ref, ba_ref, g_ref, b_ref, wb_ref,
                     bb_ref, out_ref, acc_ref, *, n_steps):
    j = pl.program_id(0)
    bvec = batch_ref[...]
    rows = jax.lax.broadcasted_iota(jnp.int32, (N_GRAPHS, bvec.shape[1]), 0)
    ph = (bvec == rows).astype(jnp.float32)

    @pl.when(j == 0)
    def _init():
        acc_ref[...] = jnp.zeros_like(acc_ref)

    acc_ref[...] += jax.lax.dot(ph, h_ref[...],
                                preferred_element_type=jnp.float32)

    @pl.when(j == n_steps - 1)
    def _fin():
        hh = jax.lax.dot(acc_ref[...], wa_ref[...],
                         preferred_element_type=jnp.float32) + ba_ref[...]
        hh = jax.nn.relu(_ln_rows(hh, g_ref[...], b_ref[...]))
        out_ref[...] = jax.lax.dot(hh, wb_ref[...],
                                   preferred_element_type=jnp.float32) \
            + bb_ref[...]


def _pool_out(h, batch2d, p):
    block = 2048
    n_steps = N_NODES // block
    rep = lambda j: (0, 0)
    return pl.pallas_call(
        functools.partial(_pool_out_kernel, n_steps=n_steps),
        grid=(n_steps,),
        in_specs=[
            pl.BlockSpec((block, D), lambda j: (j, 0)),
            pl.BlockSpec((1, block), lambda j: (0, j)),
            pl.BlockSpec((D, D), rep),
            pl.BlockSpec((1, D), rep),
            pl.BlockSpec((1, D), rep),
            pl.BlockSpec((1, D), rep),
            pl.BlockSpec((D, 1), rep),
            pl.BlockSpec((1, 1), rep),
        ],
        out_specs=pl.BlockSpec((N_GRAPHS, 1), lambda j: (0, 0)),
        out_shape=jax.ShapeDtypeStruct((N_GRAPHS, 1), jnp.float32),
        scratch_shapes=[pltpu.VMEM((N_GRAPHS, D), jnp.float32)],
    )(h, batch2d, p['out_Wa'], p['out_ba'].reshape(1, D),
      p['out_g'].reshape(1, D), p['out_b'].reshape(1, D), p['out_Wb'],
      p['out_bb'].reshape(1, 1))


def kernel(x, pos, edge_index0, edge_index1, batch, params):
    p = params
    emb_pad = jnp.pad(p['atom_emb'], ((0, 0), (0, 128 - ATOM_VOCAB), (0, 0)))
    emb_pad = emb_pad.reshape(NUM_ATOM_FEATS * D, D)
    feats = _embed(x.astype(jnp.int32), emb_pad)

    sq = (pos ** 2).sum(-1)
    dist2 = sq[:, None] + sq[None, :] - 2.0 * (pos @ pos.T)
    neg, idx = jax.lax.top_k(-dist2, K_NN)
    d_k = jnp.maximum(-neg, 0.0).reshape(NNZ, 1)
    feats_j = feats[idx].reshape(NNZ, D)

    feats = _egnn_dense(feats, feats_j, d_k, p)

    V = edge_index0.astype(jnp.int32)
    E = edge_index1.astype(jnp.int32)
    B = jnp.zeros((N_EDGES_H, N_NODES), jnp.float32).at[E, V].add(1.0)
    Bt = jnp.zeros((N_NODES, N_EDGES_H), jnp.float32).at[V, E].add(1.0)

    x0 = feats
    h = feats
    for _ in range(N_LAYERS):
        y = _mlp2_pallas(h, p, 'c1')
        xe = _spmm(B, y, relu_out=True)
        z = _mlp2_pallas(xe, p, 'c2')
        xv = _spmm(Bt, z, relu_out=False)
        h = _mlp2_pallas(xv + x0, p, 'c3', relu_out=True)

    out = _pool_out(h, batch.astype(jnp.int32).reshape(1, N_NODES), p)
    return out.reshape(-1)


# speedup vs baseline: 1.8173x; 1.1692x over previous
import functools

import jax
import jax.numpy as jnp
from jax.experimental import pallas as pl
from jax.experimental.pallas import tpu as pltpu

N_NODES = 4096
N_EDGES_H = 2048
NNZ = 65536
N_GRAPHS = 64
D = 128
M_DIM = 16
K_NN = 16
VALID_RADIUS = 5.0
N_LAYERS = 3
NUM_ATOM_FEATS = 9

EI = 2 * D + 1
EH = EI * 2
NB = 256
EB = NB * K_NN


def _ln(x, g, b, eps=1e-5):
    mu = x.mean(-1, keepdims=True)
    var = ((x - mu) ** 2).mean(-1, keepdims=True)
    return (x - mu) / jnp.sqrt(var + eps) * g + b


KR = 256


def _knn_kernel(d2_ref, d_ref, idx_ref):
    d2 = d2_ref[...]
    col = jax.lax.broadcasted_iota(jnp.int32, (KR, N_NODES), 1)
    col16 = jax.lax.broadcasted_iota(jnp.int32, (KR, K_NN), 1)
    dacc = jnp.zeros((KR, K_NN), jnp.float32)
    iacc = jnp.zeros((KR, K_NN), jnp.int32)
    big = jnp.int32(1 << 30)
    for k in range(K_NN):
        m = d2.min(axis=1, keepdims=True)
        am = jnp.where(d2 == m, col, big).min(axis=1, keepdims=True)
        dacc = jnp.where(col16 == k, jnp.maximum(m, 0.0), dacc)
        iacc = jnp.where(col16 == k, am, iacc)
        d2 = jnp.where(col == am, jnp.float32(jnp.inf), d2)
    d_ref[...] = dacc
    idx_ref[...] = iacc


def _knn_select(dist2):
    return pl.pallas_call(
        _knn_kernel,
        grid=(N_NODES // KR,),
        in_specs=[pl.BlockSpec((KR, N_NODES), lambda i: (i, 0))],
        out_specs=[
            pl.BlockSpec((KR, K_NN), lambda i: (i, 0)),
            pl.BlockSpec((KR, K_NN), lambda i: (i, 0)),
        ],
        out_shape=[
            jax.ShapeDtypeStruct((N_NODES, K_NN), jnp.float32),
            jax.ShapeDtypeStruct((N_NODES, K_NN), jnp.int32),
        ],
    )(dist2)


def _edge_mlp_kernel(feats_ref, fj_ref, dk_ref, we1_ref, be1_ref, we2_ref,
                     be2_ref, m_ref):
    feats = feats_ref[...]
    dcol = dk_ref[...]
    fi = jnp.broadcast_to(feats[:, None, :], (NB, K_NN, D)).reshape(EB, D)
    edge_in = jnp.concatenate([fi, fj_ref[...], dcol], axis=1)
    m1 = jax.lax.dot(edge_in, we1_ref[...],
                     preferred_element_type=jnp.float32) + be1_ref[...]
    m1 = m1 * jax.nn.sigmoid(m1)
    m2 = jax.lax.dot(m1, we2_ref[...],
                     preferred_element_type=jnp.float32) + be2_ref[...]
    m_ref[...] = m2 * jax.nn.sigmoid(m2)


def _edge_mlp(feats, feats_j, d_k, p):
    rep = lambda i: (0, 0)
    return pl.pallas_call(
        _edge_mlp_kernel,
        grid=(N_NODES // NB,),
        in_specs=[
            pl.BlockSpec((NB, D), lambda i: (i, 0)),
            pl.BlockSpec((EB, D), lambda i: (i, 0)),
            pl.BlockSpec((EB, 1), lambda i: (i, 0)),
            pl.BlockSpec((EI, EH), rep),
            pl.BlockSpec((1, EH), rep),
            pl.BlockSpec((EH, M_DIM), rep),
            pl.BlockSpec((1, M_DIM), rep),
        ],
        out_specs=pl.BlockSpec((EB, M_DIM), lambda i: (i, 0)),
        out_shape=jax.ShapeDtypeStruct((NNZ, M_DIM), jnp.float32),
    )(feats, feats_j, d_k, p['We1'], p['be1'].reshape(1, EH), p['We2'],
      p['be2'].reshape(1, M_DIM))


def _node_mlp_kernel(lnf_ref, mi_ref, feats_ref, wn1_ref, bn1_ref, wn2_ref,
                     bn2_ref, out_ref):
    node_in = jnp.concatenate([lnf_ref[...], mi_ref[...]], axis=1)
    h = jax.lax.dot(node_in, wn1_ref[...],
                    preferred_element_type=jnp.float32) + bn1_ref[...]
    h = h * jax.nn.sigmoid(h)
    h = jax.lax.dot(h, wn2_ref[...],
                    preferred_element_type=jnp.float32) + bn2_ref[...]
    out_ref[...] = feats_ref[...] + h


def _node_mlp(ln_f, m_i, feats, p):
    rep = lambda i: (0, 0)
    blk = 512
    return pl.pallas_call(
        _node_mlp_kernel,
        grid=(N_NODES // blk,),
        in_specs=[
            pl.BlockSpec((blk, D), lambda i: (i, 0)),
            pl.BlockSpec((blk, M_DIM), lambda i: (i, 0)),
            pl.BlockSpec((blk, D), lambda i: (i, 0)),
            pl.BlockSpec((D + M_DIM, 2 * D), rep),
            pl.BlockSpec((1, 2 * D), rep),
            pl.BlockSpec((2 * D, D), rep),
            pl.BlockSpec((1, D), rep),
        ],
        out_specs=pl.BlockSpec((blk, D), lambda i: (i, 0)),
        out_shape=jax.ShapeDtypeStruct((N_NODES, D), jnp.float32),
    )(ln_f, m_i, feats, p['Wn1'], p['bn1'].reshape(1, 2 * D), p['Wn2'],
      p['bn2'].reshape(1, D))


def _dot_bias_kernel(x_ref, w_ref, b_ref, o_ref):
    o_ref[...] = jax.lax.dot(x_ref[...], w_ref[...],
                             preferred_element_type=jnp.float32) + b_ref[...]


def _dot_bias(x, w, b, blk=512):
    n, kd = x.shape
    od = w.shape[1]
    blk = min(blk, n)
    rep = lambda i: (0, 0)
    return pl.pallas_call(
        _dot_bias_kernel,
        grid=(n // blk,),
        in_specs=[
            pl.BlockSpec((blk, kd), lambda i: (i, 0)),
            pl.BlockSpec((kd, od), rep),
            pl.BlockSpec((1, od), rep),
        ],
        out_specs=pl.BlockSpec((blk, od), lambda i: (i, 0)),
        out_shape=jax.ShapeDtypeStruct((n, od), jnp.float32),
    )(x, w, b.reshape(1, od))


def _mlp2(x, p, nm):
    h = _dot_bias(x, p[nm + '_Wa'], p[nm + '_ba'])
    h = jax.nn.relu(_ln(h, p[nm + '_g'], p[nm + '_b']))
    return _dot_bias(h, p[nm + '_Wb'], p[nm + '_bb'])


def kernel(x, pos, edge_index0, edge_index1, batch, params):
    p = params
    emb = p['atom_emb']
    feats = jnp.zeros((x.shape[0], D), jnp.float32)
    for f in range(NUM_ATOM_FEATS):
        feats = feats + emb[f][x[:, f]]

    sq = (pos ** 2).sum(-1)
    dist2 = sq[:, None] + sq[None, :] - 2.0 * (pos @ pos.T)
    d_knn, idx = _knn_select(dist2)
    d_k = d_knn.reshape(NNZ, 1)
    feats_j = feats[idx].reshape(NNZ, D)
    m = _edge_mlp(feats, feats_j, d_k, p)
    mask = d_knn <= VALID_RADIUS ** 2
    m_i = jnp.where(mask[..., None], m.reshape(N_NODES, K_NN, M_DIM),
                    0.0).sum(axis=1)
    ln_f = _ln(feats, p['ng'], p['nb'])
    feats = _node_mlp(ln_f, m_i, feats, p)

    V = edge_index0
    E = edge_index1
    x0 = feats
    h = feats
    for _ in range(N_LAYERS):
        y = _mlp2(h, p, 'c1')
        xe = jax.nn.relu(jax.ops.segment_sum(y[V], E, num_segments=N_EDGES_H))
        z = _mlp2(xe, p, 'c2')
        xv = jax.ops.segment_sum(z[E], V, num_segments=N_NODES)
        h = jax.nn.relu(_mlp2(xv + x0, p, 'c3'))

    pooled = jax.ops.segment_sum(h, batch, num_segments=N_GRAPHS)
    out = _mlp2(pooled, p, 'out')
    return out.reshape(-1)


# speedup vs baseline: 1.8329x; 1.0086x over previous
import functools

import jax
import jax.numpy as jnp
from jax.experimental import pallas as pl
from jax.experimental.pallas import tpu as pltpu

N_NODES = 4096
N_EDGES_H = 2048
NNZ = 65536
N_GRAPHS = 64
D = 128
M_DIM = 16
K_NN = 16
VALID_RADIUS = 5.0
N_LAYERS = 3
NUM_ATOM_FEATS = 9

EI = 2 * D + 1
EH = EI * 2
NB = 256
EB = NB * K_NN


def _ln(x, g, b, eps=1e-5):
    mu = x.mean(-1, keepdims=True)
    var = ((x - mu) ** 2).mean(-1, keepdims=True)
    return (x - mu) / jnp.sqrt(var + eps) * g + b


KR = 256


def _knn_kernel(d2_ref, d_ref, idx_ref):
    d2 = d2_ref[...]
    col = jax.lax.broadcasted_iota(jnp.int32, (KR, N_NODES), 1)
    col16 = jax.lax.broadcasted_iota(jnp.int32, (KR, K_NN), 1)
    dacc = jnp.zeros((KR, K_NN), jnp.float32)
    iacc = jnp.zeros((KR, K_NN), jnp.int32)
    big = jnp.int32(1 << 30)
    for k in range(K_NN):
        m = d2.min(axis=1, keepdims=True)
        am = jnp.where(d2 == m, col, big).min(axis=1, keepdims=True)
        dacc = jnp.where(col16 == k, jnp.maximum(m, 0.0), dacc)
        iacc = jnp.where(col16 == k, am, iacc)
        d2 = jnp.where(col == am, jnp.float32(jnp.inf), d2)
    d_ref[...] = dacc
    idx_ref[...] = iacc


def _knn_select(dist2):
    return pl.pallas_call(
        _knn_kernel,
        grid=(N_NODES // KR,),
        in_specs=[pl.BlockSpec((KR, N_NODES), lambda i: (i, 0))],
        out_specs=[
            pl.BlockSpec((KR, K_NN), lambda i: (i, 0)),
            pl.BlockSpec((KR, K_NN), lambda i: (i, 0)),
        ],
        out_shape=[
            jax.ShapeDtypeStruct((N_NODES, K_NN), jnp.float32),
            jax.ShapeDtypeStruct((N_NODES, K_NN), jnp.int32),
        ],
    )(dist2)


def _edge_mlp_kernel(feats_ref, fj_ref, dk_ref, we1_ref, be1_ref, we2_ref,
                     be2_ref, m_ref):
    feats = feats_ref[...]
    dcol = dk_ref[...]
    fi = jnp.broadcast_to(feats[:, None, :], (NB, K_NN, D)).reshape(EB, D)
    edge_in = jnp.concatenate([fi, fj_ref[...], dcol], axis=1)
    m1 = jax.lax.dot(edge_in, we1_ref[...],
                     preferred_element_type=jnp.float32) + be1_ref[...]
    m1 = m1 * jax.nn.sigmoid(m1)
    m2 = jax.lax.dot(m1, we2_ref[...],
                     preferred_element_type=jnp.float32) + be2_ref[...]
    m_ref[...] = m2 * jax.nn.sigmoid(m2)


def _edge_mlp(feats, feats_j, d_k, p):
    rep = lambda i: (0, 0)
    return pl.pallas_call(
        _edge_mlp_kernel,
        grid=(N_NODES // NB,),
        in_specs=[
            pl.BlockSpec((NB, D), lambda i: (i, 0)),
            pl.BlockSpec((EB, D), lambda i: (i, 0)),
            pl.BlockSpec((EB, 1), lambda i: (i, 0)),
            pl.BlockSpec((EI, EH), rep),
            pl.BlockSpec((1, EH), rep),
            pl.BlockSpec((EH, M_DIM), rep),
            pl.BlockSpec((1, M_DIM), rep),
        ],
        out_specs=pl.BlockSpec((EB, M_DIM), lambda i: (i, 0)),
        out_shape=jax.ShapeDtypeStruct((NNZ, M_DIM), jnp.float32),
    )(feats, feats_j, d_k, p['We1'], p['be1'].reshape(1, EH), p['We2'],
      p['be2'].reshape(1, M_DIM))


def _node_mlp_kernel(lnf_ref, mi_ref, feats_ref, wn1_ref, bn1_ref, wn2_ref,
                     bn2_ref, out_ref):
    node_in = jnp.concatenate([lnf_ref[...], mi_ref[...]], axis=1)
    h = jax.lax.dot(node_in, wn1_ref[...],
                    preferred_element_type=jnp.float32) + bn1_ref[...]
    h = h * jax.nn.sigmoid(h)
    h = jax.lax.dot(h, wn2_ref[...],
                    preferred_element_type=jnp.float32) + bn2_ref[...]
    out_ref[...] = feats_ref[...] + h


def _node_mlp(ln_f, m_i, feats, p):
    rep = lambda i: (0, 0)
    blk = 512
    return pl.pallas_call(
        _node_mlp_kernel,
        grid=(N_NODES // blk,),
        in_specs=[
            pl.BlockSpec((blk, D), lambda i: (i, 0)),
            pl.BlockSpec((blk, M_DIM), lambda i: (i, 0)),
            pl.BlockSpec((blk, D), lambda i: (i, 0)),
            pl.BlockSpec((D + M_DIM, 2 * D), rep),
            pl.BlockSpec((1, 2 * D), rep),
            pl.BlockSpec((2 * D, D), rep),
            pl.BlockSpec((1, D), rep),
        ],
        out_specs=pl.BlockSpec((blk, D), lambda i: (i, 0)),
        out_shape=jax.ShapeDtypeStruct((N_NODES, D), jnp.float32),
    )(ln_f, m_i, feats, p['Wn1'], p['bn1'].reshape(1, 2 * D), p['Wn2'],
      p['bn2'].reshape(1, D))


def _dot_bias_kernel(x_ref, w_ref, b_ref, o_ref):
    o_ref[...] = jax.lax.dot(x_ref[...], w_ref[...],
                             preferred_element_type=jnp.float32) + b_ref[...]


def _dot_bias(x, w, b, blk=512):
    n, kd = x.shape
    od = w.shape[1]
    blk = min(blk, n)
    rep = lambda i: (0, 0)
    return pl.pallas_call(
        _dot_bias_kernel,
        grid=(n // blk,),
        in_specs=[
            pl.BlockSpec((blk, kd), lambda i: (i, 0)),
            pl.BlockSpec((kd, od), rep),
            pl.BlockSpec((1, od), rep),
        ],
        out_specs=pl.BlockSpec((blk, od), lambda i: (i, 0)),
        out_shape=jax.ShapeDtypeStruct((n, od), jnp.float32),
    )(x, w, b.reshape(1, od))


def _mlp2(x, p, nm):
    h = _dot_bias(x, p[nm + '_Wa'], p[nm + '_ba'])
    h = jax.nn.relu(_ln(h, p[nm + '_g'], p[nm + '_b']))
    return _dot_bias(h, p[nm + '_Wb'], p[nm + '_bb'])


def _take_sorted(a, idx):
    dn = jax.lax.GatherDimensionNumbers(
        offset_dims=(1,), collapsed_slice_dims=(0,), start_index_map=(0,))
    return jax.lax.gather(
        a, idx[:, None], dimension_numbers=dn, slice_sizes=(1, a.shape[1]),
        indices_are_sorted=True,
        mode=jax.lax.GatherScatterMode.PROMISE_IN_BOUNDS)


def kernel(x, pos, edge_index0, edge_index1, batch, params):
    p = params
    emb = p['atom_emb']
    feats = jnp.zeros((x.shape[0], D), jnp.float32)
    for f in range(NUM_ATOM_FEATS):
        feats = feats + emb[f][x[:, f]]

    sq = (pos ** 2).sum(-1)
    dist2 = sq[:, None] + sq[None, :] - 2.0 * (pos @ pos.T)
    d_knn, idx = _knn_select(dist2)
    d_k = d_knn.reshape(NNZ, 1)
    feats_j = feats[idx].reshape(NNZ, D)
    m = _edge_mlp(feats, feats_j, d_k, p)
    mask = d_knn <= VALID_RADIUS ** 2
    m_i = jnp.where(mask[..., None], m.reshape(N_NODES, K_NN, M_DIM),
                    0.0).sum(axis=1)
    ln_f = _ln(feats, p['ng'], p['nb'])
    feats = _node_mlp(ln_f, m_i, feats, p)

    V = edge_index0
    E = edge_index1
    x0 = feats
    h = feats
    for _ in range(N_LAYERS):
        y = _mlp2(h, p, 'c1')
        xe = jax.nn.relu(jax.ops.segment_sum(
            y[V], E, num_segments=N_EDGES_H, indices_are_sorted=True))
        z = _mlp2(xe, p, 'c2')
        xv = jax.ops.segment_sum(_take_sorted(z, E), V, num_segments=N_NODES)
        h = jax.nn.relu(_mlp2(xv + x0, p, 'c3'))

    pooled = jax.ops.segment_sum(h, batch, num_segments=N_GRAPHS,
                                 indices_are_sorted=True)
    out = _mlp2(pooled, p, 'out')
    return out.reshape(-1)


# speedup vs baseline: 2.8036x; 1.5296x over previous
import functools

import jax
import jax.numpy as jnp
from jax.experimental import pallas as pl
from jax.experimental.pallas import tpu as pltpu
from jax.experimental.pallas import tpu_sc as plsc

N_NODES = 4096
N_EDGES_H = 2048
NNZ = 65536
N_GRAPHS = 64
D = 128
M_DIM = 16
K_NN = 16
VALID_RADIUS = 5.0
N_LAYERS = 3
NUM_ATOM_FEATS = 9

EI = 2 * D + 1
EH = EI * 2
NB = 256
EB = NB * K_NN


def _ln(x, g, b, eps=1e-5):
    mu = x.mean(-1, keepdims=True)
    var = ((x - mu) ** 2).mean(-1, keepdims=True)
    return (x - mu) / jnp.sqrt(var + eps) * g + b


KR = 256


def _knn_kernel(d2_ref, d_ref, idx_ref):
    d2 = d2_ref[...]
    col = jax.lax.broadcasted_iota(jnp.int32, (KR, N_NODES), 1)
    col16 = jax.lax.broadcasted_iota(jnp.int32, (KR, K_NN), 1)
    dacc = jnp.zeros((KR, K_NN), jnp.float32)
    iacc = jnp.zeros((KR, K_NN), jnp.int32)
    big = jnp.int32(1 << 30)
    for k in range(K_NN):
        m = d2.min(axis=1, keepdims=True)
        am = jnp.where(d2 == m, col, big).min(axis=1, keepdims=True)
        dacc = jnp.where(col16 == k, jnp.maximum(m, 0.0), dacc)
        iacc = jnp.where(col16 == k, am, iacc)
        d2 = jnp.where(col == am, jnp.float32(jnp.inf), d2)
    d_ref[...] = dacc
    idx_ref[...] = iacc


def _knn_select(dist2):
    return pl.pallas_call(
        _knn_kernel,
        grid=(N_NODES // KR,),
        in_specs=[pl.BlockSpec((KR, N_NODES), lambda i: (i, 0))],
        out_specs=[
            pl.BlockSpec((KR, K_NN), lambda i: (i, 0)),
            pl.BlockSpec((KR, K_NN), lambda i: (i, 0)),
        ],
        out_shape=[
            jax.ShapeDtypeStruct((N_NODES, K_NN), jnp.float32),
            jax.ShapeDtypeStruct((N_NODES, K_NN), jnp.int32),
        ],
    )(dist2)


def _edge_mlp_kernel(feats_ref, fj_ref, dk_ref, we1_ref, be1_ref, we2_ref,
                     be2_ref, m_ref):
    feats = feats_ref[...]
    dcol = dk_ref[...]
    fi = jnp.broadcast_to(feats[:, None, :], (NB, K_NN, D)).reshape(EB, D)
    edge_in = jnp.concatenate([fi, fj_ref[...], dcol], axis=1)
    m1 = jax.lax.dot(edge_in, we1_ref[...],
                     preferred_element_type=jnp.float32) + be1_ref[...]
    m1 = m1 * jax.nn.sigmoid(m1)
    m2 = jax.lax.dot(m1, we2_ref[...],
                     preferred_element_type=jnp.float32) + be2_ref[...]
    m_ref[...] = m2 * jax.nn.sigmoid(m2)


def _edge_mlp(feats, feats_j, d_k, p):
    rep = lambda i: (0, 0)
    return pl.pallas_call(
        _edge_mlp_kernel,
        grid=(N_NODES // NB,),
        in_specs=[
            pl.BlockSpec((NB, D), lambda i: (i, 0)),
            pl.BlockSpec((EB, D), lambda i: (i, 0)),
            pl.BlockSpec((EB, 1), lambda i: (i, 0)),
            pl.BlockSpec((EI, EH), rep),
            pl.BlockSpec((1, EH), rep),
            pl.BlockSpec((EH, M_DIM), rep),
            pl.BlockSpec((1, M_DIM), rep),
        ],
        out_specs=pl.BlockSpec((EB, M_DIM), lambda i: (i, 0)),
        out_shape=jax.ShapeDtypeStruct((NNZ, M_DIM), jnp.float32),
    )(feats, feats_j, d_k, p['We1'], p['be1'].reshape(1, EH), p['We2'],
      p['be2'].reshape(1, M_DIM))


def _node_mlp_kernel(lnf_ref, mi_ref, feats_ref, wn1_ref, bn1_ref, wn2_ref,
                     bn2_ref, out_ref):
    node_in = jnp.concatenate([lnf_ref[...], mi_ref[...]], axis=1)
    h = jax.lax.dot(node_in, wn1_ref[...],
                    preferred_element_type=jnp.float32) + bn1_ref[...]
    h = h * jax.nn.sigmoid(h)
    h = jax.lax.dot(h, wn2_ref[...],
                    preferred_element_type=jnp.float32) + bn2_ref[...]
    out_ref[...] = feats_ref[...] + h


def _node_mlp(ln_f, m_i, feats, p):
    rep = lambda i: (0, 0)
    blk = 512
    return pl.pallas_call(
        _node_mlp_kernel,
        grid=(N_NODES // blk,),
        in_specs=[
            pl.BlockSpec((blk, D), lambda i: (i, 0)),
            pl.BlockSpec((blk, M_DIM), lambda i: (i, 0)),
            pl.BlockSpec((blk, D), lambda i: (i, 0)),
            pl.BlockSpec((D + M_DIM, 2 * D), rep),
            pl.BlockSpec((1, 2 * D), rep),
            pl.BlockSpec((2 * D, D), rep),
            pl.BlockSpec((1, D), rep),
        ],
        out_specs=pl.BlockSpec((blk, D), lambda i: (i, 0)),
        out_shape=jax.ShapeDtypeStruct((N_NODES, D), jnp.float32),
    )(ln_f, m_i, feats, p['Wn1'], p['bn1'].reshape(1, 2 * D), p['Wn2'],
      p['bn2'].reshape(1, D))


def _dot_bias_kernel(x_ref, w_ref, b_ref, o_ref):
    o_ref[...] = jax.lax.dot(x_ref[...], w_ref[...],
                             preferred_element_type=jnp.float32) + b_ref[...]


def _dot_bias(x, w, b, blk=512):
    n, kd = x.shape
    od = w.shape[1]
    blk = min(blk, n)
    rep = lambda i: (0, 0)
    return pl.pallas_call(
        _dot_bias_kernel,
        grid=(n // blk,),
        in_specs=[
            pl.BlockSpec((blk, kd), lambda i: (i, 0)),
            pl.BlockSpec((kd, od), rep),
            pl.BlockSpec((1, od), rep),
        ],
        out_specs=pl.BlockSpec((blk, od), lambda i: (i, 0)),
        out_shape=jax.ShapeDtypeStruct((n, od), jnp.float32),
    )(x, w, b.reshape(1, od))


def _mlp2(x, p, nm):
    h = _dot_bias(x, p[nm + '_Wa'], p[nm + '_ba'])
    h = jax.nn.relu(_ln(h, p[nm + '_g'], p[nm + '_b']))
    return _dot_bias(h, p[nm + '_Wb'], p[nm + '_bb'])


def _sc_gather(table, idx):
    n = idx.shape[0]
    d = table.shape[1]
    nw = 32
    b_per_w = n // nw
    ch = 512
    mesh = plsc.VectorSubcoreMesh(core_axis_name="c", subcore_axis_name="s")

    @functools.partial(
        pl.kernel, mesh=mesh,
        out_type=jax.ShapeDtypeStruct((n, d), jnp.float32),
        scratch_types=[
            pltpu.VMEM((ch,), jnp.int32),
            pltpu.VMEM((ch, d), jnp.float32),
            pltpu.SemaphoreType.DMA,
        ],
    )
    def k(table_hbm, idx_hbm, out_hbm, idx_v, rows_v, sem):
        wid = jax.lax.axis_index("s") * 2 + jax.lax.axis_index("c")
        base = wid * b_per_w
        for c in range(b_per_w // ch):
            off = base + c * ch
            pltpu.sync_copy(idx_hbm.at[pl.ds(off, ch)], idx_v)
            pltpu.async_copy(table_hbm.at[idx_v], rows_v, sem).wait()
            pltpu.sync_copy(rows_v, out_hbm.at[pl.ds(off, ch)])

    return k(table, idx)


def _take_sorted(a, idx):
    dn = jax.lax.GatherDimensionNumbers(
        offset_dims=(1,), collapsed_slice_dims=(0,), start_index_map=(0,))
    return jax.lax.gather(
        a, idx[:, None], dimension_numbers=dn, slice_sizes=(1, a.shape[1]),
        indices_are_sorted=True,
        mode=jax.lax.GatherScatterMode.PROMISE_IN_BOUNDS)


def kernel(x, pos, edge_index0, edge_index1, batch, params):
    p = params
    emb = p['atom_emb']
    feats = jnp.zeros((x.shape[0], D), jnp.float32)
    for f in range(NUM_ATOM_FEATS):
        feats = feats + emb[f][x[:, f]]

    sq = (pos ** 2).sum(-1)
    dist2 = sq[:, None] + sq[None, :] - 2.0 * (pos @ pos.T)
    d_knn, idx = _knn_select(dist2)
    d_k = d_knn.reshape(NNZ, 1)
    feats_j = _sc_gather(feats, idx.reshape(NNZ))
    m = _edge_mlp(feats, feats_j, d_k, p)
    mask = d_knn <= VALID_RADIUS ** 2
    m_i = jnp.where(mask[..., None], m.reshape(N_NODES, K_NN, M_DIM),
                    0.0).sum(axis=1)
    ln_f = _ln(feats, p['ng'], p['nb'])
    feats = _node_mlp(ln_f, m_i, feats, p)

    V = edge_index0.astype(jnp.int32)
    E = edge_index1.astype(jnp.int32)
    x0 = feats
    h = feats
    for _ in range(N_LAYERS):
        y = _mlp2(h, p, 'c1')
        xe = jax.nn.relu(jax.ops.segment_sum(
            _sc_gather(y, V), E, num_segments=N_EDGES_H,
            indices_are_sorted=True))
        z = _mlp2(xe, p, 'c2')
        xv = jax.ops.segment_sum(_sc_gather(z, E), V, num_segments=N_NODES)
        h = jax.nn.relu(_mlp2(xv + x0, p, 'c3'))

    pooled = jax.ops.segment_sum(h, batch, num_segments=N_GRAPHS,
                                 indices_are_sorted=True)
    out = _mlp2(pooled, p, 'out')
    return out.reshape(-1)


# speedup vs baseline: 2.8044x; 1.0003x over previous
import functools

import jax
import jax.numpy as jnp
from jax.experimental import pallas as pl
from jax.experimental.pallas import tpu as pltpu
from jax.experimental.pallas import tpu_sc as plsc

N_NODES = 4096
N_EDGES_H = 2048
NNZ = 65536
N_GRAPHS = 64
D = 128
M_DIM = 16
K_NN = 16
VALID_RADIUS = 5.0
N_LAYERS = 3
NUM_ATOM_FEATS = 9

EI = 2 * D + 1
EH = EI * 2
NB = 256
EB = NB * K_NN


def _ln(x, g, b, eps=1e-5):
    mu = x.mean(-1, keepdims=True)
    var = ((x - mu) ** 2).mean(-1, keepdims=True)
    return (x - mu) / jnp.sqrt(var + eps) * g + b


KR = 256


def _knn_kernel(d2_ref, d_ref, idx_ref):
    d2 = d2_ref[...]
    col = jax.lax.broadcasted_iota(jnp.int32, (KR, N_NODES), 1)
    col16 = jax.lax.broadcasted_iota(jnp.int32, (KR, K_NN), 1)
    dacc = jnp.zeros((KR, K_NN), jnp.float32)
    iacc = jnp.zeros((KR, K_NN), jnp.int32)
    big = jnp.int32(1 << 30)
    for k in range(K_NN):
        m = d2.min(axis=1, keepdims=True)
        am = jnp.where(d2 == m, col, big).min(axis=1, keepdims=True)
        dacc = jnp.where(col16 == k, jnp.maximum(m, 0.0), dacc)
        iacc = jnp.where(col16 == k, am, iacc)
        d2 = jnp.where(col == am, jnp.float32(jnp.inf), d2)
    d_ref[...] = dacc
    idx_ref[...] = iacc


def _knn_select(dist2):
    return pl.pallas_call(
        _knn_kernel,
        grid=(N_NODES // KR,),
        in_specs=[pl.BlockSpec((KR, N_NODES), lambda i: (i, 0))],
        out_specs=[
            pl.BlockSpec((KR, K_NN), lambda i: (i, 0)),
            pl.BlockSpec((KR, K_NN), lambda i: (i, 0)),
        ],
        out_shape=[
            jax.ShapeDtypeStruct((N_NODES, K_NN), jnp.float32),
            jax.ShapeDtypeStruct((N_NODES, K_NN), jnp.int32),
        ],
    )(dist2)


def _edge_mlp_kernel(feats_ref, fj_ref, dk_ref, we1_ref, be1_ref, we2_ref,
                     be2_ref, m_ref):
    feats = feats_ref[...]
    dcol = dk_ref[...]
    fi = jnp.broadcast_to(feats[:, None, :], (NB, K_NN, D)).reshape(EB, D)
    edge_in = jnp.concatenate([fi, fj_ref[...], dcol], axis=1)
    m1 = jax.lax.dot(edge_in, we1_ref[...],
                     preferred_element_type=jnp.float32) + be1_ref[...]
    m1 = m1 * jax.nn.sigmoid(m1)
    m2 = jax.lax.dot(m1, we2_ref[...],
                     preferred_element_type=jnp.float32) + be2_ref[...]
    m_ref[...] = m2 * jax.nn.sigmoid(m2)


def _edge_mlp(feats, feats_j, d_k, p):
    rep = lambda i: (0, 0)
    return pl.pallas_call(
        _edge_mlp_kernel,
        grid=(N_NODES // NB,),
        in_specs=[
            pl.BlockSpec((NB, D), lambda i: (i, 0)),
            pl.BlockSpec((EB, D), lambda i: (i, 0)),
            pl.BlockSpec((EB, 1), lambda i: (i, 0)),
            pl.BlockSpec((EI, EH), rep),
            pl.BlockSpec((1, EH), rep),
            pl.BlockSpec((EH, M_DIM), rep),
            pl.BlockSpec((1, M_DIM), rep),
        ],
        out_specs=pl.BlockSpec((EB, M_DIM), lambda i: (i, 0)),
        out_shape=jax.ShapeDtypeStruct((NNZ, M_DIM), jnp.float32),
    )(feats, feats_j, d_k, p['We1'], p['be1'].reshape(1, EH), p['We2'],
      p['be2'].reshape(1, M_DIM))


def _node_mlp_kernel(lnf_ref, mi_ref, feats_ref, wn1_ref, bn1_ref, wn2_ref,
                     bn2_ref, out_ref):
    node_in = jnp.concatenate([lnf_ref[...], mi_ref[...]], axis=1)
    h = jax.lax.dot(node_in, wn1_ref[...],
                    preferred_element_type=jnp.float32) + bn1_ref[...]
    h = h * jax.nn.sigmoid(h)
    h = jax.lax.dot(h, wn2_ref[...],
                    preferred_element_type=jnp.float32) + bn2_ref[...]
    out_ref[...] = feats_ref[...] + h


def _node_mlp(ln_f, m_i, feats, p):
    rep = lambda i: (0, 0)
    blk = 512
    return pl.pallas_call(
        _node_mlp_kernel,
        grid=(N_NODES // blk,),
        in_specs=[
            pl.BlockSpec((blk, D), lambda i: (i, 0)),
            pl.BlockSpec((blk, M_DIM), lambda i: (i, 0)),
            pl.BlockSpec((blk, D), lambda i: (i, 0)),
            pl.BlockSpec((D + M_DIM, 2 * D), rep),
            pl.BlockSpec((1, 2 * D), rep),
            pl.BlockSpec((2 * D, D), rep),
            pl.BlockSpec((1, D), rep),
        ],
        out_specs=pl.BlockSpec((blk, D), lambda i: (i, 0)),
        out_shape=jax.ShapeDtypeStruct((N_NODES, D), jnp.float32),
    )(ln_f, m_i, feats, p['Wn1'], p['bn1'].reshape(1, 2 * D), p['Wn2'],
      p['bn2'].reshape(1, D))


def _dot_bias_kernel(x_ref, w_ref, b_ref, o_ref):
    o_ref[...] = jax.lax.dot(x_ref[...], w_ref[...],
                             preferred_element_type=jnp.float32) + b_ref[...]


def _dot_bias(x, w, b, blk=512):
    n, kd = x.shape
    od = w.shape[1]
    blk = min(blk, n)
    rep = lambda i: (0, 0)
    return pl.pallas_call(
        _dot_bias_kernel,
        grid=(n // blk,),
        in_specs=[
            pl.BlockSpec((blk, kd), lambda i: (i, 0)),
            pl.BlockSpec((kd, od), rep),
            pl.BlockSpec((1, od), rep),
        ],
        out_specs=pl.BlockSpec((blk, od), lambda i: (i, 0)),
        out_shape=jax.ShapeDtypeStruct((n, od), jnp.float32),
    )(x, w, b.reshape(1, od))


def _mlp2(x, p, nm):
    h = _dot_bias(x, p[nm + '_Wa'], p[nm + '_ba'])
    h = jax.nn.relu(_ln(h, p[nm + '_g'], p[nm + '_b']))
    return _dot_bias(h, p[nm + '_Wb'], p[nm + '_bb'])


def _sc_gather(table, idx):
    n = idx.shape[0]
    d = table.shape[1]
    nw = 32
    b_per_w = n // nw
    ch = 512
    mesh = plsc.VectorSubcoreMesh(core_axis_name="c", subcore_axis_name="s")

    @functools.partial(
        pl.kernel, mesh=mesh,
        out_type=jax.ShapeDtypeStruct((n, d), jnp.float32),
        scratch_types=[
            pltpu.VMEM((ch,), jnp.int32),
            pltpu.VMEM((ch, d), jnp.float32),
            pltpu.SemaphoreType.DMA,
        ],
    )
    def k(table_hbm, idx_hbm, out_hbm, idx_v, rows_v, sem):
        wid = jax.lax.axis_index("s") * 2 + jax.lax.axis_index("c")
        base = wid * b_per_w
        for c in range(b_per_w // ch):
            off = base + c * ch
            pltpu.sync_copy(idx_hbm.at[pl.ds(off, ch)], idx_v)
            pltpu.async_copy(table_hbm.at[idx_v], rows_v, sem).wait()
            pltpu.sync_copy(rows_v, out_hbm.at[pl.ds(off, ch)])

    return k(table, idx)


def kernel(x, pos, edge_index0, edge_index1, batch, params):
    p = params
    emb = p['atom_emb']
    feats = jnp.zeros((x.shape[0], D), jnp.float32)
    for f in range(NUM_ATOM_FEATS):
        feats = feats + emb[f][x[:, f]]

    sq = (pos ** 2).sum(-1)
    dist2 = sq[:, None] + sq[None, :] - 2.0 * (pos @ pos.T)
    d_knn, idx = _knn_select(dist2)
    d_k = d_knn.reshape(NNZ, 1)
    feats_j = _sc_gather(feats, idx.reshape(NNZ))
    m = _edge_mlp(feats, feats_j, d_k, p)
    mask = d_knn <= VALID_RADIUS ** 2
    m_i = jnp.where(mask[..., None], m.reshape(N_NODES, K_NN, M_DIM),
                    0.0).sum(axis=1)
    ln_f = _ln(feats, p['ng'], p['nb'])
    feats = _node_mlp(ln_f, m_i, feats, p)

    V = edge_index0.astype(jnp.int32)
    E = edge_index1.astype(jnp.int32)
    x0 = feats
    h = feats
    for _ in range(N_LAYERS):
        y = _mlp2(h, p, 'c1')
        xe = jax.nn.relu(jax.ops.segment_sum(
            _sc_gather(y, V), E, num_segments=N_EDGES_H,
            indices_are_sorted=True))
        z = _mlp2(xe, p, 'c2')
        xv = jax.ops.segment_sum(_sc_gather(z, E), V, num_segments=N_NODES)
        h = jax.nn.relu(_mlp2(xv + x0, p, 'c3'))

    pooled = jax.ops.segment_sum(h, batch, num_segments=N_GRAPHS,
                                 indices_are_sorted=True)
    out = _mlp2(pooled, p, 'out')
    return out.reshape(-1)
